# Initial kernel scaffold; baseline (speedup 1.0000x reference)
import jax
import jax.numpy as jnp
from jax.experimental import pallas as pl


def kernel(user, item, tag, tag_type, table_u, table_i, table_t):
    raise NotImplementedError("write your pallas kernel here")


# speedup vs baseline: 3.4277x; 3.4277x over previous
import functools

import jax
import jax.numpy as jnp
from jax import lax
from jax.experimental import pallas as pl
from jax.experimental.pallas import tpu as pltpu
from jax.experimental.pallas import tpu_sc as plsc

B = 16384
C = 50
D = 32

NC = 2
NS = 16
NW = NC * NS
BPW = B // NW
CH = 32
NCHUNK = BPW // CH
TAG_SUB = 100
SUBS = CH * C // TAG_SUB


def _sc_kernel(user_h, item_h, tag_h, tu_h, ti_h, tt_h, out_h,
               uidx_v, iidx_v, tidx_v, urows_v, irows_v, trows_v,
               out_v, sem):
    wid = lax.axis_index("s") * NC + lax.axis_index("c")
    iota = lax.iota(jnp.int32, 16)

    @pl.loop(0, NCHUNK)
    def _chunk(ch):
        b0 = pl.multiple_of(wid * BPW + ch * CH, CH)
        trow0 = pl.multiple_of(b0 * C // TAG_SUB, 16)

        pltpu.sync_copy(user_h.at[pl.ds(b0, CH)], uidx_v)
        pltpu.sync_copy(item_h.at[pl.ds(b0, CH)], iidx_v)
        pltpu.sync_copy(tag_h.at[pl.ds(trow0, SUBS)], tidx_v)

        copies = []
        copies.append(pltpu.async_copy(tu_h.at[uidx_v], urows_v, sem))
        copies.append(pltpu.async_copy(ti_h.at[iidx_v], irows_v, sem))
        for j in range(SUBS):
            copies.append(
                pltpu.async_copy(tt_h.at[tidx_v.at[j]],
                                 trows_v.at[pl.ds(j * TAG_SUB, TAG_SUB)],
                                 sem))
        for cp in copies:
            cp.wait()

        @pl.loop(0, CH)
        def _row(b):
            s_lo = urows_v[b, pl.ds(0, 16)] + irows_v[b, pl.ds(0, 16)]
            s_hi = urows_v[b, pl.ds(16, 16)] + irows_v[b, pl.ds(16, 16)]
            for g in range(4):
                nlc = 16 if g < 3 else C - 48
                acc = jnp.zeros((16,), jnp.float32)
                for lc in range(nlc):
                    p = b * C + g * 16 + lc
                    t_lo = trows_v[p, pl.ds(0, 16)]
                    t_hi = trows_v[p, pl.ds(16, 16)]
                    r = jnp.sum(s_lo * t_lo + s_hi * t_hi)
                    acc = jnp.where(iota == lc, r, acc)
                plsc.store_scatter(out_v, [iota + (b * C + g * 16)], acc,
                                   mask=iota < nlc)

        pltpu.sync_copy(out_v, out_h.at[pl.ds(b0 * C, CH * C)])


def kernel(user, item, tag, tag_type, table_u, table_i, table_t):
    del tag_type
    user = user.astype(jnp.int32)
    item = item.astype(jnp.int32)
    tag2 = tag.astype(jnp.int32).reshape(B * C // TAG_SUB, TAG_SUB)

    mesh = plsc.VectorSubcoreMesh(core_axis_name="c", subcore_axis_name="s")
    run = functools.partial(
        pl.kernel,
        out_type=jax.ShapeDtypeStruct((B * C,), jnp.float32),
        mesh=mesh,
        compiler_params=pltpu.CompilerParams(needs_layout_passes=False,
                                             use_tc_tiling_on_sc=False),
        scratch_types=[
            pltpu.VMEM((CH,), jnp.int32),
            pltpu.VMEM((CH,), jnp.int32),
            pltpu.VMEM((SUBS, TAG_SUB), jnp.int32),
            pltpu.VMEM((CH, D), jnp.float32),
            pltpu.VMEM((CH, D), jnp.float32),
            pltpu.VMEM((CH * C, D), jnp.float32),
            pltpu.VMEM((CH * C,), jnp.float32),
            pltpu.SemaphoreType.DMA,
        ],
    )(_sc_kernel)
    return run(user, item, tag2, table_u, table_i, table_t).reshape(B, C)
